# Initial kernel scaffold; baseline (speedup 1.0000x reference)
import jax
import jax.numpy as jnp
from jax.experimental import pallas as pl


def kernel(x, edge_index, p, W_init, gru_Wi, gru_Wh, gru_bi, gru_bh, lin_w, lin_b):
    raise NotImplementedError("write your pallas kernel here")


# speedup vs baseline: 17.9745x; 17.9745x over previous
import functools

import jax
import jax.numpy as jnp
from jax import lax
from jax.experimental import pallas as pl
from jax.experimental.pallas import tpu as pltpu
from jax.experimental.pallas import tpu_sc as plsc

NC, NS, L = 2, 16, 16
NW = NC * NS
CHUNK = 128


def _sc_mesh():
    return plsc.VectorSubcoreMesh(
        core_axis_name="c", subcore_axis_name="s", num_cores=NC, num_subcores=NS)


def _make_degree_kernel(chunks, nodes_pad):
    rows_per_tile = nodes_pad // NS

    @functools.partial(
        pl.kernel,
        out_type=jax.ShapeDtypeStruct((NC, nodes_pad), jnp.float32),
        mesh=_sc_mesh(),
        scratch_types=[
            pltpu.VMEM((chunks, CHUNK), jnp.int32),
            pltpu.VMEM((CHUNK,), jnp.float32),
            pltpu.VMEM((rows_per_tile,), jnp.float32),
            pltpu.VMEM_SHARED((nodes_pad,), jnp.float32),
            pltpu.SemaphoreType.DMA,
        ],
    )
    def deg_kernel(dst_hbm, deg_hbm, dst_v, ones_v, stage_v, acc_sh, sem):
        del sem
        cid = lax.axis_index("c")
        sid = lax.axis_index("s")
        wid = sid * NC + cid
        for t in range(CHUNK // L):
            ones_v[pl.ds(t * L, L)] = jnp.ones((L,), jnp.float32)

        def zero_body(i, _):
            stage_v[pl.ds(i * L, L)] = jnp.zeros((L,), jnp.float32)
            return 0

        lax.fori_loop(0, rows_per_tile // L, zero_body, 0)
        base = sid * rows_per_tile
        pltpu.sync_copy(stage_v, acc_sh.at[pl.ds(base, rows_per_tile)])
        plsc.subcore_barrier()
        pltpu.sync_copy(dst_hbm.at[wid], dst_v)

        def body(j, _):
            pltpu.sync_copy(ones_v, acc_sh.at[dst_v.at[j]], add=True)
            return 0

        lax.fori_loop(0, chunks, body, 0)
        plsc.subcore_barrier()
        pltpu.sync_copy(acc_sh.at[pl.ds(base, rows_per_tile)], stage_v)
        pltpu.sync_copy(stage_v, deg_hbm.at[cid, pl.ds(base, rows_per_tile)])

    return deg_kernel


def _make_aggregate_kernel(chunks, nodes_pad, c):
    rows_per_tile = nodes_pad // NS

    @functools.partial(
        pl.kernel,
        out_type=jax.ShapeDtypeStruct((NC, nodes_pad, c), jnp.float32),
        mesh=_sc_mesh(),
        scratch_types=[
            pltpu.VMEM((chunks, CHUNK), jnp.int32),
            pltpu.VMEM((chunks, CHUNK), jnp.int32),
            pltpu.VMEM((CHUNK, c), jnp.float32),
            pltpu.VMEM_SHARED((nodes_pad, c), jnp.float32),
            pltpu.SemaphoreType.DMA,
        ],
    )
    def agg_kernel(xs_hbm, src_hbm, dst_hbm, part_hbm,
                   src_v, dst_v, rows_v, acc_sh, sem):
        cid = lax.axis_index("c")
        sid = lax.axis_index("s")
        wid = sid * NC + cid

        def zero_body(i, _):
            for t in range(c // L):
                rows_v[i, pl.ds(t * L, L)] = jnp.zeros((L,), jnp.float32)
            return 0

        lax.fori_loop(0, CHUNK, zero_body, 0)
        base = sid * rows_per_tile
        for t in range(rows_per_tile // CHUNK):
            pltpu.sync_copy(rows_v, acc_sh.at[pl.ds(base + t * CHUNK, CHUNK)])
        plsc.subcore_barrier()
        pltpu.sync_copy(src_hbm.at[wid], src_v)
        pltpu.sync_copy(dst_hbm.at[wid], dst_v)

        def body(j, _):
            pltpu.async_copy(xs_hbm.at[src_v.at[j]], rows_v, sem).wait()
            pltpu.sync_copy(rows_v, acc_sh.at[dst_v.at[j]], add=True)
            return 0

        lax.fori_loop(0, chunks, body, 0)
        plsc.subcore_barrier()
        for t in range(rows_per_tile // CHUNK):
            pltpu.sync_copy(acc_sh.at[pl.ds(base + t * CHUNK, CHUNK)], rows_v)
            pltpu.sync_copy(rows_v, part_hbm.at[cid, pl.ds(base + t * CHUNK, CHUNK)])

    return agg_kernel


def _dense_body(n_real, nodes_pad, c, k,
                x_ref, xr_ref, p_ref, pr_ref, w0_ref, gwi_ref, gwh_ref,
                gbi_ref, gbh_ref, deg_ref, xs_ref, dis_ref, xt_ref, z2d_ref):
    nb = nodes_pad // CHUNK
    z_row = lax.dot_general(pr_ref[...], xr_ref[...], (((1,), (1,)), ((), ())))
    for r2 in range(nb):
        z2d_ref[r2:r2 + 1, :] = z_row[:, r2 * CHUNK:(r2 + 1) * CHUNK]
    z2d = z2d_ref[...]
    flat = (lax.broadcasted_iota(jnp.int32, (nb, CHUNK), 0) * CHUNK
            + lax.broadcasted_iota(jnp.int32, (nb, CHUNK), 1))
    z2d = jnp.where(flat >= n_real, -jnp.inf, z2d)
    pn = jnp.sqrt(jnp.sum(p_ref[0, :] * p_ref[0, :]))
    inv_pn = 1.0 / pn

    def tk_body(j, z):
        m = jnp.max(z)
        sel = jnp.where(z == m, flat, jnp.int32(1 << 30))
        fidx = jnp.min(sel)
        val = jnp.tanh(m * inv_pn)
        xt_ref[pl.ds(j, 1), :] = x_ref[pl.ds(fidx, 1), :] * val
        return jnp.where(sel == fidx, -jnp.inf, z)

    lax.fori_loop(0, k, tk_body, z2d)
    x_tilde = xt_ref[...]

    dims = (((1,), (1,)), ((), ()))
    gi = lax.dot_general(x_tilde, gwi_ref[...], dims) + gbi_ref[...]
    gh = lax.dot_general(w0_ref[...], gwh_ref[...], dims) + gbh_ref[...]
    i_r, i_z, i_n = gi[:, :c], gi[:, c:2 * c], gi[:, 2 * c:]
    h_r, h_z, h_n = gh[:, :c], gh[:, c:2 * c], gh[:, 2 * c:]
    r = jax.nn.sigmoid(i_r + h_r)
    zz = jax.nn.sigmoid(i_z + h_z)
    nn_ = jnp.tanh(i_n + r * h_n)
    w_ev = (1.0 - zz) * nn_ + zz * w0_ref[...]

    xw = lax.dot_general(x_ref[...], w_ev, (((1,), (0,)), ((), ())))
    deg = deg_ref[0] + deg_ref[1] + 1.0
    dis = lax.rsqrt(deg)
    xs_ref[...] = xw * dis
    dis_ref[...] = dis


def _head_body(part_ref, xs_ref, dis_ref, lw_ref, lb_ref, y_ref):
    t = part_ref[0] + part_ref[1] + xs_ref[...]
    h = jnp.maximum(t * dis_ref[...], 0.0)
    y = jnp.sum(h * lw_ref[...], axis=1, keepdims=True)
    y_ref[...] = y + lb_ref[0, 0]


def kernel(x, edge_index, p, W_init, gru_Wi, gru_Wh, gru_bi, gru_bh, lin_w, lin_b):
    n, c = x.shape
    k = c
    e = edge_index.shape[1]
    nodes_pad = ((n + 1 + NS * CHUNK - 1) // (NS * CHUNK)) * (NS * CHUNK)
    chunks = (e + NW * CHUNK - 1) // (NW * CHUNK)
    e_pad = NW * chunks * CHUNK

    src = edge_index[0].astype(jnp.int32)
    dst = edge_index[1].astype(jnp.int32)
    pad = jnp.full((e_pad - e,), n, jnp.int32)
    src_p = jnp.concatenate([src, pad]).reshape(NW, chunks, CHUNK)
    dst_p = jnp.concatenate([dst, pad]).reshape(NW, chunks, CHUNK)

    xpad = jnp.pad(x, ((0, nodes_pad - n), (0, 0)))
    xr = xpad.astype(jnp.bfloat16).astype(jnp.float32)
    p_round = p.astype(jnp.bfloat16).astype(jnp.float32)

    deg_parts = _make_degree_kernel(chunks, nodes_pad)(dst_p)
    deg3 = deg_parts.reshape(NC, nodes_pad, 1)

    xs, dis = pl.pallas_call(
        functools.partial(_dense_body, n, nodes_pad, c, k),
        out_shape=[
            jax.ShapeDtypeStruct((nodes_pad, c), jnp.float32),
            jax.ShapeDtypeStruct((nodes_pad, 1), jnp.float32),
        ],
        scratch_shapes=[pltpu.VMEM((k, c), jnp.float32),
                        pltpu.VMEM((nodes_pad // CHUNK, CHUNK), jnp.float32)],
    )(xpad, xr, p.reshape(1, c), p_round.reshape(1, c), W_init, gru_Wi, gru_Wh,
      gru_bi.reshape(1, 3 * c), gru_bh.reshape(1, 3 * c), deg3)

    parts = _make_aggregate_kernel(chunks, nodes_pad, c)(xs, src_p, dst_p)

    y = pl.pallas_call(
        _head_body,
        out_shape=jax.ShapeDtypeStruct((nodes_pad, 1), jnp.float32),
    )(parts, xs, dis, lin_w, lin_b.reshape(1, 1))
    return y[:n]
